# baseline (speedup 1.0000x reference)
import functools

import jax
import jax.numpy as jnp
from jax import lax
from jax.experimental import pallas as pl
from jax.experimental.pallas import tpu as pltpu
from jax.experimental.pallas import tpu_sc as plsc

VOCAB = 8192
D = 128
B = 8
T = 512
N_POOL = 1000000
K_WIN = 64
LOOPS = 4
THRESH = 0.99
NTOK = B * T


def _ln(x, g, b):
    m = x.mean(-1, keepdims=True)
    v = ((x - m) ** 2).mean(-1, keepdims=True)
    return (x - m) / jnp.sqrt(v + 1e-6) * g + b


def _sc_embed_gather(ids_flat, table):
    info = plsc.get_sparse_core_info()
    NC, NS = info.num_cores, info.num_subcores
    NW = NC * NS
    rows_per_w = NTOK // NW

    mesh = plsc.VectorSubcoreMesh(core_axis_name="c", subcore_axis_name="s")

    @functools.partial(
        pl.kernel,
        mesh=mesh,
        out_type=jax.ShapeDtypeStruct((NTOK, D), jnp.float32),
        scratch_types=[
            pltpu.VMEM((rows_per_w,), jnp.int32),
            pltpu.VMEM((rows_per_w, D), jnp.float32),
            pltpu.SemaphoreType.DMA,
        ],
    )
    def k(ids_hbm, table_hbm, out_hbm, idx_v, rows_v, sem):
        wid = lax.axis_index("s") * NC + lax.axis_index("c")
        base = wid * rows_per_w
        pltpu.sync_copy(ids_hbm.at[pl.ds(base, rows_per_w)], idx_v)
        pltpu.async_copy(table_hbm.at[idx_v], rows_v, sem).wait()
        pltpu.sync_copy(rows_v, out_hbm.at[pl.ds(base, rows_per_w)])

    return k(ids_flat, table)


def _ffn_kernel(x_ref, w1_ref, b1_ref, w2_ref, b2_ref, g1_ref, be1_ref, out_ref):
    x = x_ref[...]
    h = jax.nn.gelu(jnp.dot(x, w1_ref[...], preferred_element_type=jnp.float32)
                    + b1_ref[...])
    ff = jnp.dot(h, w2_ref[...], preferred_element_type=jnp.float32) + b2_ref[...]
    out_ref[...] = _ln(x + ff, g1_ref[...], be1_ref[...])


def _controller_ffn(x, W1, b1, W2, b2, g1, be1):
    TILE = 512
    grid = (NTOK // TILE,)
    return pl.pallas_call(
        _ffn_kernel,
        grid=grid,
        in_specs=[
            pl.BlockSpec((TILE, D), lambda i: (i, 0)),
            pl.BlockSpec((D, 4 * D), lambda i: (0, 0)),
            pl.BlockSpec((1, 4 * D), lambda i: (0, 0)),
            pl.BlockSpec((4 * D, D), lambda i: (0, 0)),
            pl.BlockSpec((1, D), lambda i: (0, 0)),
            pl.BlockSpec((1, D), lambda i: (0, 0)),
            pl.BlockSpec((1, D), lambda i: (0, 0)),
        ],
        out_specs=pl.BlockSpec((TILE, D), lambda i: (i, 0)),
        out_shape=jax.ShapeDtypeStruct((NTOK, D), jnp.float32),
    )(x, W1, b1.reshape(1, -1), W2, b2.reshape(1, -1),
      g1.reshape(1, -1), be1.reshape(1, -1))


def _recur_kernel(ids_ref, embed_ref, w1_ref, b1_ref, w2_ref, b2_ref,
                  g1_ref, be1_ref,
                  widx_ref, bidx_ref, pool_ref, wi1_ref, bi1_ref,
                  wi2_ref, bi2_ref, g2_ref, be2_ref, wh_ref, bh_ref,
                  retr_ref, starts_ref,
                  vecs_scr, start_scr, xl_scr, sem):
    for b in range(B):
        pltpu.make_async_copy(
            embed_ref.at[pl.ds(ids_ref[b, T - 1], 1)],
            xl_scr.at[pl.ds(b, 1)], sem).start()
    for b in range(B):
        pltpu.make_async_copy(
            embed_ref.at[pl.ds(0, 1)], xl_scr.at[pl.ds(b, 1)], sem).wait()
    xl = xl_scr[...]
    ff = jnp.dot(jax.nn.gelu(jnp.dot(xl, w1_ref[...],
                                     preferred_element_type=jnp.float32)
                             + b1_ref[...]), w2_ref[...],
                 preferred_element_type=jnp.float32) + b2_ref[...]
    state = _ln(xl + ff, g1_ref[...], be1_ref[...])
    hp = jnp.zeros((B, 1), jnp.float32)
    hm = jnp.zeros((B, 1), jnp.float32)
    kwin = lax.broadcasted_iota(jnp.int32, (B, K_WIN), 1).astype(jnp.float32)
    bd_mask = (lax.broadcasted_iota(jnp.int32, (B, B * K_WIN), 1) // K_WIN
               == lax.broadcasted_iota(jnp.int32, (B, B * K_WIN), 0))
    starts_acc = jnp.zeros((B, 128), jnp.int32)
    lane = lax.broadcasted_iota(jnp.int32, (B, 128), 1)

    for s in range(LOOPS):
        raw = jnp.dot(state, widx_ref[...], preferred_element_type=jnp.float32) \
            + bidx_ref[...]
        mu = jax.nn.sigmoid(raw[:, 0:1])
        sg = jax.nn.softplus(raw[:, 1:2]) + 1e-3
        startf = jnp.floor(mu * float(N_POOL - K_WIN))
        start = jnp.clip(startf.astype(jnp.int32), 0, N_POOL - K_WIN)
        start_scr[...] = jnp.broadcast_to(start, (B, 128))
        starts_acc = jnp.where(lane == s, start, starts_acc)

        for b in range(B):
            s0 = start_scr[b, 0]
            pltpu.make_async_copy(
                pool_ref.at[pl.ds(s0, K_WIN)],
                vecs_scr.at[pl.ds(b * K_WIN, K_WIN)],
                sem,
            ).start()
        for b in range(B):
            pltpu.make_async_copy(
                pool_ref.at[pl.ds(0, K_WIN)],
                vecs_scr.at[pl.ds(b * K_WIN, K_WIN)],
                sem,
            ).wait()

        pos = (start.astype(jnp.float32) + kwin) / float(N_POOL)
        wlog = -((pos - mu) ** 2) / (2.0 * sg * sg)
        w = jax.nn.softmax(wlog, axis=-1)
        w_bd = jnp.where(bd_mask, jnp.tile(w, (1, B)), 0.0)
        retrieved = jnp.dot(w_bd, vecs_scr[...],
                            preferred_element_type=jnp.float32)
        retr_ref[:, s, :] = retrieved

        combined = jnp.concatenate([state, retrieved], axis=-1)
        h1 = jax.nn.gelu(jnp.dot(combined, wi1_ref[...],
                                 preferred_element_type=jnp.float32) + bi1_ref[...])
        integ = _ln(jnp.dot(h1, wi2_ref[...],
                            preferred_element_type=jnp.float32) + bi2_ref[...],
                    g2_ref[...], be2_ref[...])
        cand = state + integ
        p = jax.nn.sigmoid(jnp.dot(cand, wh_ref[...],
                                   preferred_element_type=jnp.float32) + bh_ref[...])
        nhp = hp + p * (1.0 - hm)
        nhm = jnp.where(nhp >= THRESH, 1.0, hm)
        state = (1.0 - hm) * cand + hm * state
        hp, hm = nhp, nhm

    starts_ref[...] = starts_acc


def _recurrence(ids_last, embed, W1, b1, W2, b2, g1, be1, Widx, bidx,
                pool_table, Wi1, bi1, Wi2, bi2, g2, be2, Wh, bh):
    return pl.pallas_call(
        _recur_kernel,
        in_specs=[
            pl.BlockSpec(memory_space=pl.ANY) if i in (1, 10)
            else pl.BlockSpec()
            for i in range(19)
        ],
        out_specs=[pl.BlockSpec(), pl.BlockSpec()],
        out_shape=[
            jax.ShapeDtypeStruct((B, LOOPS, D), jnp.float32),
            jax.ShapeDtypeStruct((B, 128), jnp.int32),
        ],
        scratch_shapes=[
            pltpu.VMEM((B * K_WIN, D), jnp.float32),
            pltpu.VMEM((B, 128), jnp.int32),
            pltpu.VMEM((B, D), jnp.float32),
            pltpu.SemaphoreType.DMA,
        ],
    )(ids_last, embed, W1, b1.reshape(1, -1), W2, b2.reshape(1, -1),
      g1.reshape(1, -1), be1.reshape(1, -1),
      Widx, bidx.reshape(1, -1), pool_table, Wi1, bi1.reshape(1, -1),
      Wi2, bi2.reshape(1, -1), g2.reshape(1, -1), be2.reshape(1, -1),
      Wh, bh.reshape(1, -1))


def _integ_dec_kernel(x_ref, w1_ref, b1_ref, w2_ref, b2_ref, g1_ref, be1_ref,
                      retr_ref, wi1_ref, bi1_ref, wi2_ref, bi2_ref,
                      g2_ref, be2_ref, wh_ref, bh_ref, wdec_ref, bdec_ref,
                      out_ref, wdec_scr):
    @pl.when(jnp.logical_and(pl.program_id(0) == 0, pl.program_id(1) == 0))
    def _():
        wdec_scr[...] = wdec_ref[...].astype(jnp.bfloat16)

    x = x_ref[0]
    ff = jnp.dot(jax.nn.gelu(jnp.dot(x, w1_ref[...],
                                     preferred_element_type=jnp.float32)
                             + b1_ref[...]), w2_ref[...],
                 preferred_element_type=jnp.float32) + b2_ref[...]
    state = _ln(x + ff, g1_ref[...], be1_ref[...])
    retr_all = retr_ref[0]
    hp = jnp.zeros((T, 1), jnp.float32)
    hm = jnp.zeros((T, 1), jnp.float32)
    for s in range(LOOPS):
        retr = jnp.broadcast_to(retr_all[s:s + 1, :], (T, D))
        combined = jnp.concatenate([state, retr], axis=-1)
        h1 = jax.nn.gelu(jnp.dot(combined, wi1_ref[...],
                                 preferred_element_type=jnp.float32)
                         + bi1_ref[...])
        integ = _ln(jnp.dot(h1, wi2_ref[...],
                            preferred_element_type=jnp.float32)
                    + bi2_ref[...], g2_ref[...], be2_ref[...])
        cand = state + integ
        p = jax.nn.sigmoid(jnp.dot(cand, wh_ref[...],
                                   preferred_element_type=jnp.float32)
                           + bh_ref[...])
        nhp = hp + p * (1.0 - hm)
        nhm = jnp.where(nhp >= THRESH, 1.0, hm)
        state = (1.0 - hm) * cand + hm * state
        hp, hm = nhp, nhm
    v = pl.program_id(1)
    out_ref[0] = jnp.dot(state.astype(jnp.bfloat16),
                         wdec_scr[:, pl.ds(v * VTILE, VTILE)],
                         preferred_element_type=jnp.float32) + bdec_ref[...]


VTILE = 4096


def _integrate_decode(x, W1, b1, W2, b2, g1, be1, retr,
                      Wi1, bi1, Wi2, bi2, g2, be2, Wh, bh, Wdec, bdec):
    grid = (B, VOCAB // VTILE)
    x3 = x.reshape(B, T, D)
    return pl.pallas_call(
        _integ_dec_kernel,
        grid=grid,
        in_specs=[
            pl.BlockSpec((1, T, D), lambda b, v: (b, 0, 0)),
            pl.BlockSpec((D, 4 * D), lambda b, v: (0, 0)),
            pl.BlockSpec((1, 4 * D), lambda b, v: (0, 0)),
            pl.BlockSpec((4 * D, D), lambda b, v: (0, 0)),
            pl.BlockSpec((1, D), lambda b, v: (0, 0)),
            pl.BlockSpec((1, D), lambda b, v: (0, 0)),
            pl.BlockSpec((1, D), lambda b, v: (0, 0)),
            pl.BlockSpec((1, LOOPS, D), lambda b, v: (b, 0, 0)),
            pl.BlockSpec((2 * D, D), lambda b, v: (0, 0)),
            pl.BlockSpec((1, D), lambda b, v: (0, 0)),
            pl.BlockSpec((D, D), lambda b, v: (0, 0)),
            pl.BlockSpec((1, D), lambda b, v: (0, 0)),
            pl.BlockSpec((1, D), lambda b, v: (0, 0)),
            pl.BlockSpec((1, D), lambda b, v: (0, 0)),
            pl.BlockSpec((D, 1), lambda b, v: (0, 0)),
            pl.BlockSpec((1, 1), lambda b, v: (0, 0)),
            pl.BlockSpec((D, VOCAB), lambda b, v: (0, 0)),
            pl.BlockSpec((1, VTILE), lambda b, v: (0, v)),
        ],
        out_specs=pl.BlockSpec((1, T, VTILE), lambda b, v: (b, 0, v)),
        out_shape=jax.ShapeDtypeStruct((B, T, VOCAB), jnp.float32),
        scratch_shapes=[pltpu.VMEM((D, VOCAB), jnp.bfloat16)],
        compiler_params=pltpu.CompilerParams(
            dimension_semantics=("arbitrary", "arbitrary")),
    )(x3, W1, b1.reshape(1, -1), W2, b2.reshape(1, -1),
      g1.reshape(1, -1), be1.reshape(1, -1),
      retr, Wi1, bi1.reshape(1, -1),
      Wi2, bi2.reshape(1, -1),
      g2.reshape(1, -1), be2.reshape(1, -1), Wh, bh.reshape(1, -1),
      Wdec, bdec.reshape(1, -1))


def kernel(input_ids, embed, W1, b1, W2, b2, g1, be1, Wdec, bdec, Widx, bidx,
           pool_table, Wi1, bi1, Wi2, bi2, g2, be2, Wh, bh):
    ids_flat = input_ids.reshape(-1).astype(jnp.int32)
    x = _sc_embed_gather(ids_flat, embed)
    ids_2d = input_ids.astype(jnp.int32)
    retr, starts = _recurrence(ids_2d, embed, W1, b1, W2, b2, g1, be1,
                               Widx, bidx, pool_table,
                               Wi1, bi1, Wi2, bi2, g2, be2, Wh, bh)
    logits = _integrate_decode(x, W1, b1, W2, b2, g1, be1, retr,
                               Wi1, bi1, Wi2, bi2, g2, be2,
                               Wh, bh, Wdec, bdec)
    all_indices = starts[:, :LOOPS]
    return logits, all_indices


# speedup vs baseline: 1.2602x; 1.2602x over previous
import functools

import jax
import jax.numpy as jnp
from jax import lax
from jax.experimental import pallas as pl
from jax.experimental.pallas import tpu as pltpu
from jax.experimental.pallas import tpu_sc as plsc

VOCAB = 8192
D = 128
B = 8
T = 512
N_POOL = 1000000
K_WIN = 64
LOOPS = 4
THRESH = 0.99
NTOK = B * T


def _ln(x, g, b):
    m = x.mean(-1, keepdims=True)
    v = ((x - m) ** 2).mean(-1, keepdims=True)
    return (x - m) / jnp.sqrt(v + 1e-6) * g + b


def _sc_embed_gather(ids_flat, table):
    info = plsc.get_sparse_core_info()
    NC, NS = info.num_cores, info.num_subcores
    NW = NC * NS
    rows_per_w = NTOK // NW

    mesh = plsc.VectorSubcoreMesh(core_axis_name="c", subcore_axis_name="s")

    @functools.partial(
        pl.kernel,
        mesh=mesh,
        out_type=jax.ShapeDtypeStruct((NTOK, D), jnp.float32),
        scratch_types=[
            pltpu.VMEM((rows_per_w,), jnp.int32),
            pltpu.VMEM((rows_per_w, D), jnp.float32),
            pltpu.SemaphoreType.DMA,
        ],
    )
    def k(ids_hbm, table_hbm, out_hbm, idx_v, rows_v, sem):
        wid = lax.axis_index("s") * NC + lax.axis_index("c")
        base = wid * rows_per_w
        pltpu.sync_copy(ids_hbm.at[pl.ds(base, rows_per_w)], idx_v)
        pltpu.async_copy(table_hbm.at[idx_v], rows_v, sem).wait()
        pltpu.sync_copy(rows_v, out_hbm.at[pl.ds(base, rows_per_w)])

    return k(ids_flat, table)


def _recur_kernel(ids_ref, embed_ref, w1_ref, b1_ref, w2_ref, b2_ref,
                  g1_ref, be1_ref,
                  widx_ref, bidx_ref, pool_ref, wi1_ref, bi1_ref,
                  wi2_ref, bi2_ref, g2_ref, be2_ref, wh_ref, bh_ref,
                  retr_ref, starts_ref,
                  vecs_scr, start_scr, xl_scr, sem):
    for b in range(B):
        pltpu.make_async_copy(
            embed_ref.at[pl.ds(ids_ref[b, T - 1], 1)],
            xl_scr.at[pl.ds(b, 1)], sem).start()
    for b in range(B):
        pltpu.make_async_copy(
            embed_ref.at[pl.ds(0, 1)], xl_scr.at[pl.ds(b, 1)], sem).wait()
    xl = xl_scr[...]
    ff = jnp.dot(jax.nn.gelu(jnp.dot(xl, w1_ref[...],
                                     preferred_element_type=jnp.float32)
                             + b1_ref[...]), w2_ref[...],
                 preferred_element_type=jnp.float32) + b2_ref[...]
    state = _ln(xl + ff, g1_ref[...], be1_ref[...])
    hp = jnp.zeros((B, 1), jnp.float32)
    hm = jnp.zeros((B, 1), jnp.float32)
    kwin = lax.broadcasted_iota(jnp.int32, (B, K_WIN), 1).astype(jnp.float32)
    bd_mask = (lax.broadcasted_iota(jnp.int32, (B, B * K_WIN), 1) // K_WIN
               == lax.broadcasted_iota(jnp.int32, (B, B * K_WIN), 0))
    starts_acc = jnp.zeros((B, 128), jnp.int32)
    lane = lax.broadcasted_iota(jnp.int32, (B, 128), 1)

    for s in range(LOOPS):
        raw = jnp.dot(state, widx_ref[...], preferred_element_type=jnp.float32) \
            + bidx_ref[...]
        mu = jax.nn.sigmoid(raw[:, 0:1])
        sg = jax.nn.softplus(raw[:, 1:2]) + 1e-3
        startf = jnp.floor(mu * float(N_POOL - K_WIN))
        start = jnp.clip(startf.astype(jnp.int32), 0, N_POOL - K_WIN)
        start_scr[...] = jnp.broadcast_to(start, (B, 128))
        starts_acc = jnp.where(lane == s, start, starts_acc)

        for b in range(B):
            s0 = start_scr[b, 0]
            pltpu.make_async_copy(
                pool_ref.at[pl.ds(s0, K_WIN)],
                vecs_scr.at[pl.ds(b * K_WIN, K_WIN)],
                sem,
            ).start()
        for b in range(B):
            pltpu.make_async_copy(
                pool_ref.at[pl.ds(0, K_WIN)],
                vecs_scr.at[pl.ds(b * K_WIN, K_WIN)],
                sem,
            ).wait()

        pos = (start.astype(jnp.float32) + kwin) / float(N_POOL)
        wlog = -((pos - mu) ** 2) / (2.0 * sg * sg)
        w = jax.nn.softmax(wlog, axis=-1)
        w_bd = jnp.where(bd_mask, jnp.tile(w, (1, B)), 0.0)
        retrieved = jnp.dot(w_bd, vecs_scr[...],
                            preferred_element_type=jnp.float32)
        retr_ref[:, s, :] = retrieved

        combined = jnp.concatenate([state, retrieved], axis=-1)
        h1 = jax.nn.gelu(jnp.dot(combined, wi1_ref[...],
                                 preferred_element_type=jnp.float32) + bi1_ref[...])
        integ = _ln(jnp.dot(h1, wi2_ref[...],
                            preferred_element_type=jnp.float32) + bi2_ref[...],
                    g2_ref[...], be2_ref[...])
        cand = state + integ
        p = jax.nn.sigmoid(jnp.dot(cand, wh_ref[...],
                                   preferred_element_type=jnp.float32) + bh_ref[...])
        nhp = hp + p * (1.0 - hm)
        nhm = jnp.where(nhp >= THRESH, 1.0, hm)
        state = (1.0 - hm) * cand + hm * state
        hp, hm = nhp, nhm

    starts_ref[...] = starts_acc


def _recurrence(ids_last, embed, W1, b1, W2, b2, g1, be1, Widx, bidx,
                pool_table, Wi1, bi1, Wi2, bi2, g2, be2, Wh, bh):
    return pl.pallas_call(
        _recur_kernel,
        in_specs=[
            pl.BlockSpec(memory_space=pl.ANY) if i in (1, 10)
            else pl.BlockSpec()
            for i in range(19)
        ],
        out_specs=[pl.BlockSpec(), pl.BlockSpec()],
        out_shape=[
            jax.ShapeDtypeStruct((B, LOOPS, D), jnp.float32),
            jax.ShapeDtypeStruct((B, 128), jnp.int32),
        ],
        scratch_shapes=[
            pltpu.VMEM((B * K_WIN, D), jnp.float32),
            pltpu.VMEM((B, 128), jnp.int32),
            pltpu.VMEM((B, D), jnp.float32),
            pltpu.SemaphoreType.DMA,
        ],
    )(ids_last, embed, W1, b1.reshape(1, -1), W2, b2.reshape(1, -1),
      g1.reshape(1, -1), be1.reshape(1, -1),
      Widx, bidx.reshape(1, -1), pool_table, Wi1, bi1.reshape(1, -1),
      Wi2, bi2.reshape(1, -1), g2.reshape(1, -1), be2.reshape(1, -1),
      Wh, bh.reshape(1, -1))


def _integ_dec_kernel(x_ref, w1_ref, b1_ref, w2_ref, b2_ref, g1_ref, be1_ref,
                      retr_ref, wi1_ref, bi1_ref, wi2_ref, bi2_ref,
                      g2_ref, be2_ref, wh_ref, bh_ref, wdec_ref, bdec_ref,
                      out_ref, wdec_scr):
    @pl.when(pl.program_id(0) == 0)
    def _():
        wdec_scr[...] = wdec_ref[...].astype(jnp.bfloat16)

    x = x_ref[0]
    ff = jnp.dot(jax.nn.gelu(jnp.dot(x, w1_ref[...],
                                     preferred_element_type=jnp.float32)
                             + b1_ref[...]), w2_ref[...],
                 preferred_element_type=jnp.float32) + b2_ref[...]
    state = _ln(x + ff, g1_ref[...], be1_ref[...])
    retr_all = retr_ref[0]
    hp = jnp.zeros((T, 1), jnp.float32)
    hm = jnp.zeros((T, 1), jnp.float32)
    for s in range(LOOPS):
        retr = jnp.broadcast_to(retr_all[s:s + 1, :], (T, D))
        combined = jnp.concatenate([state, retr], axis=-1)
        h1 = jax.nn.gelu(jnp.dot(combined, wi1_ref[...],
                                 preferred_element_type=jnp.float32)
                         + bi1_ref[...])
        integ = _ln(jnp.dot(h1, wi2_ref[...],
                            preferred_element_type=jnp.float32)
                    + bi2_ref[...], g2_ref[...], be2_ref[...])
        cand = state + integ
        p = jax.nn.sigmoid(jnp.dot(cand, wh_ref[...],
                                   preferred_element_type=jnp.float32)
                           + bh_ref[...])
        nhp = hp + p * (1.0 - hm)
        nhm = jnp.where(nhp >= THRESH, 1.0, hm)
        state = (1.0 - hm) * cand + hm * state
        hp, hm = nhp, nhm
    out_ref[0] = jnp.dot(state.astype(jnp.bfloat16), wdec_scr[...],
                         preferred_element_type=jnp.float32) + bdec_ref[...]


def _integrate_decode(x, W1, b1, W2, b2, g1, be1, retr,
                      Wi1, bi1, Wi2, bi2, g2, be2, Wh, bh, Wdec, bdec):
    grid = (B,)
    x3 = x.reshape(B, T, D)
    return pl.pallas_call(
        _integ_dec_kernel,
        grid=grid,
        in_specs=[
            pl.BlockSpec((1, T, D), lambda b: (b, 0, 0)),
            pl.BlockSpec((D, 4 * D), lambda b: (0, 0)),
            pl.BlockSpec((1, 4 * D), lambda b: (0, 0)),
            pl.BlockSpec((4 * D, D), lambda b: (0, 0)),
            pl.BlockSpec((1, D), lambda b: (0, 0)),
            pl.BlockSpec((1, D), lambda b: (0, 0)),
            pl.BlockSpec((1, D), lambda b: (0, 0)),
            pl.BlockSpec((1, LOOPS, D), lambda b: (b, 0, 0)),
            pl.BlockSpec((2 * D, D), lambda b: (0, 0)),
            pl.BlockSpec((1, D), lambda b: (0, 0)),
            pl.BlockSpec((D, D), lambda b: (0, 0)),
            pl.BlockSpec((1, D), lambda b: (0, 0)),
            pl.BlockSpec((1, D), lambda b: (0, 0)),
            pl.BlockSpec((1, D), lambda b: (0, 0)),
            pl.BlockSpec((D, 1), lambda b: (0, 0)),
            pl.BlockSpec((1, 1), lambda b: (0, 0)),
            pl.BlockSpec((D, VOCAB), lambda b: (0, 0)),
            pl.BlockSpec((1, VOCAB), lambda b: (0, 0)),
        ],
        out_specs=pl.BlockSpec((1, T, VOCAB), lambda b: (b, 0, 0)),
        out_shape=jax.ShapeDtypeStruct((B, T, VOCAB), jnp.float32),
        scratch_shapes=[pltpu.VMEM((D, VOCAB), jnp.bfloat16)],
        compiler_params=pltpu.CompilerParams(
            dimension_semantics=("arbitrary",)),
    )(x3, W1, b1.reshape(1, -1), W2, b2.reshape(1, -1),
      g1.reshape(1, -1), be1.reshape(1, -1),
      retr, Wi1, bi1.reshape(1, -1),
      Wi2, bi2.reshape(1, -1),
      g2.reshape(1, -1), be2.reshape(1, -1), Wh, bh.reshape(1, -1),
      Wdec, bdec.reshape(1, -1))


def kernel(input_ids, embed, W1, b1, W2, b2, g1, be1, Wdec, bdec, Widx, bidx,
           pool_table, Wi1, bi1, Wi2, bi2, g2, be2, Wh, bh):
    ids_flat = input_ids.reshape(-1).astype(jnp.int32)
    x = _sc_embed_gather(ids_flat, embed)
    ids_2d = input_ids.astype(jnp.int32)
    retr, starts = _recurrence(ids_2d, embed, W1, b1, W2, b2, g1, be1,
                               Widx, bidx, pool_table,
                               Wi1, bi1, Wi2, bi2, g2, be2, Wh, bh)
    logits = _integrate_decode(x, W1, b1, W2, b2, g1, be1, retr,
                               Wi1, bi1, Wi2, bi2, g2, be2,
                               Wh, bh, Wdec, bdec)
    all_indices = starts[:, :LOOPS]
    return logits, all_indices
